# baseline (speedup 1.0000x reference)
import functools

import jax
import jax.numpy as jnp
from jax import lax
from jax.experimental import pallas as pl
from jax.experimental.pallas import tpu as pltpu
from jax.experimental.pallas import tpu_sc as plsc

NUM_LODS = 10
TEX_H = 512
TEX_W = 512
NUM_CHANNELS = 11

NC = 2
NS = 16
NW = NC * NS
L = 16

CHUNK = 2048
SUB = 128
NSUB = CHUNK // SUB
DPAD = 16


def _tex_kernel_body(table_hbm, bi_hbm, out_hbm, slab_v, idx_v, rows_v, sem):
    wid = lax.axis_index("s") * NC + lax.axis_index("c")
    batch = out_hbm.shape[0]
    bpw = batch // NW
    iota = lax.iota(jnp.int32, L)

    def chunk_body(k, carry):
        base = wid * bpw + k * CHUNK
        pltpu.sync_copy(bi_hbm.at[pl.ds(base * 3, CHUNK * 3)], slab_v)

        def jloop(j, c):
            for l in range(SUB // L):
                off = j * SUB + l * L
                p = (off + iota) * 3
                ys = plsc.load_gather(slab_v, [p])
                xs = plsc.load_gather(slab_v, [p + 1])
                lods = plsc.load_gather(slab_v, [p + 2])
                sy = lax.shift_right_logical(ys, lods)
                sx = lax.shift_right_logical(xs, lods)
                idx = (lods << 18) + (sy << 9) + sx
                idx_v[j, l * L:(l + 1) * L] = idx
            return c

        lax.fori_loop(0, NSUB, jloop, 0)

        cps = [
            pltpu.async_copy(
                table_hbm.at[idx_v.at[j]],
                rows_v.at[pl.ds(j * SUB, SUB)],
                sem,
            )
            for j in range(NSUB)
        ]
        for cp in cps:
            cp.wait()
        pltpu.sync_copy(rows_v, out_hbm.at[pl.ds(base, CHUNK)])
        return carry

    lax.fori_loop(0, bpw // CHUNK, chunk_body, 0)


def _make_tex_gather(batch):
    mesh = plsc.VectorSubcoreMesh(
        core_axis_name="c", subcore_axis_name="s", num_cores=NC, num_subcores=NS
    )
    return functools.partial(
        pl.kernel,
        out_type=jax.ShapeDtypeStruct((batch, DPAD), jnp.float32),
        mesh=mesh,
        scratch_types=[
            pltpu.VMEM((CHUNK * 3,), jnp.int32),
            pltpu.VMEM((NSUB, SUB), jnp.int32),
            pltpu.VMEM((CHUNK, DPAD), jnp.float32),
            pltpu.SemaphoreType.DMA,
        ],
        compiler_params=pltpu.CompilerParams(
            needs_layout_passes=False, use_tc_tiling_on_sc=False
        ),
    )(_tex_kernel_body)


def kernel(lod_cache, batch_index):
    batch = batch_index.shape[0]
    table = lod_cache.reshape(NUM_LODS * TEX_H * TEX_W, NUM_CHANNELS)
    table = jnp.pad(table, ((0, 0), (0, DPAD - NUM_CHANNELS)))
    bi = batch_index.astype(jnp.int32).reshape(-1)
    out = _make_tex_gather(batch)(table, bi)
    return out[:, :NUM_CHANNELS]


# speedup vs baseline: 1.3121x; 1.3121x over previous
import functools

import jax
import jax.numpy as jnp
from jax import lax
from jax.experimental import pallas as pl
from jax.experimental.pallas import tpu as pltpu
from jax.experimental.pallas import tpu_sc as plsc

NUM_LODS = 10
TEX_H = 512
TEX_W = 512
NUM_CHANNELS = 11

NC = 2
NS = 16
NW = NC * NS
L = 16

CHUNK = 2048
SUB = 128
NSUB = CHUNK // SUB
DPAD = 16


def _tex_kernel_body(table_hbm, bi_hbm, out_hbm, slab_v, idx_v, rows_v, sem):
    wid = lax.axis_index("s") * NC + lax.axis_index("c")
    batch = out_hbm.shape[0]
    bpw = batch // NW
    iota = lax.iota(jnp.int32, L)

    def chunk_body(k, carry):
        base = wid * bpw + k * CHUNK
        pltpu.sync_copy(bi_hbm.at[pl.ds(base * 3, CHUNK * 3)], slab_v)

        def jloop(j, c):
            for l in range(SUB // L):
                off = j * SUB + l * L
                p = (off + iota) * 3
                ys = plsc.load_gather(slab_v, [p])
                xs = plsc.load_gather(slab_v, [p + 1])
                lods = plsc.load_gather(slab_v, [p + 2])
                sy = lax.shift_right_logical(ys, lods)
                sx = lax.shift_right_logical(xs, lods)
                t = (1 << 20) - lax.shift_right_logical(
                    jnp.full((L,), 1 << 20, jnp.int32), 2 * lods
                )
                base_row = t * jnp.int32(-1431655765)
                idx = base_row + lax.shift_left(sy, 9 - lods) + sx
                idx_v[j, l * L:(l + 1) * L] = idx
            return c

        lax.fori_loop(0, NSUB, jloop, 0)

        cps = [
            pltpu.async_copy(
                table_hbm.at[idx_v.at[j]],
                rows_v.at[pl.ds(j * SUB, SUB)],
                sem,
            )
            for j in range(NSUB)
        ]
        for cp in cps:
            cp.wait()
        pltpu.sync_copy(rows_v, out_hbm.at[pl.ds(base, CHUNK)])
        return carry

    lax.fori_loop(0, bpw // CHUNK, chunk_body, 0)


def _make_tex_gather(batch):
    mesh = plsc.VectorSubcoreMesh(
        core_axis_name="c", subcore_axis_name="s", num_cores=NC, num_subcores=NS
    )
    return functools.partial(
        pl.kernel,
        out_type=jax.ShapeDtypeStruct((batch, DPAD), jnp.float32),
        mesh=mesh,
        scratch_types=[
            pltpu.VMEM((CHUNK * 3,), jnp.int32),
            pltpu.VMEM((NSUB, SUB), jnp.int32),
            pltpu.VMEM((CHUNK, DPAD), jnp.float32),
            pltpu.SemaphoreType.DMA,
        ],
        compiler_params=pltpu.CompilerParams(
            needs_layout_passes=False, use_tc_tiling_on_sc=False
        ),
    )(_tex_kernel_body)


def kernel(lod_cache, batch_index):
    batch = batch_index.shape[0]
    parts = [
        lax.slice(
            lod_cache,
            (l, 0, 0, 0),
            (l + 1, TEX_H >> l, TEX_W >> l, NUM_CHANNELS),
        ).reshape(-1, NUM_CHANNELS)
        for l in range(NUM_LODS)
    ]
    table = jnp.concatenate(parts, axis=0)
    nrows = table.shape[0]
    rpad = (-nrows) % 8
    table = jnp.pad(table, ((0, rpad), (0, DPAD - NUM_CHANNELS)))
    bi = batch_index.astype(jnp.int32).reshape(-1)
    out = _make_tex_gather(batch)(table, bi)
    return out[:, :NUM_CHANNELS]
